# baseline (speedup 1.0000x reference)
import functools

import jax
import jax.numpy as jnp
from jax import lax
from jax.experimental import pallas as pl
from jax.experimental.pallas import tpu as pltpu
from jax.experimental.pallas import tpu_sc as plsc

N_FACTORS = 64
BATCH = 16384
IDX_CHUNK = 128

_info = plsc.get_sparse_core_info()
_NC, _NS, _L = _info.num_cores, _info.num_subcores, _info.num_lanes
_NW = _NC * _NS
_B_PER_W = BATCH // _NW
_N_CHUNKS = _B_PER_W // IDX_CHUNK
_N_GROUPS = _B_PER_W // _L


def _bpr_body(xr, user_t, item_t, out, idx_v, ru, ri, rj, out_v, sem):
    wid = lax.axis_index("s") * _NC + lax.axis_index("c")

    pltpu.sync_copy(xr.at[0, pl.ds(wid * _N_CHUNKS, _N_CHUNKS)], idx_v.at[0])
    pltpu.sync_copy(xr.at[1, pl.ds(wid * _N_CHUNKS, _N_CHUNKS)], idx_v.at[1])
    pltpu.sync_copy(xr.at[2, pl.ds(wid * _N_CHUNKS, _N_CHUNKS)], idx_v.at[2])

    copies = []
    for j in range(_N_CHUNKS):
        dst = pl.ds(j * IDX_CHUNK, IDX_CHUNK)
        copies.append(pltpu.async_copy(user_t.at[idx_v.at[0, j]], ru.at[dst], sem))
        copies.append(pltpu.async_copy(item_t.at[idx_v.at[1, j]], ri.at[dst], sem))
        copies.append(pltpu.async_copy(item_t.at[idx_v.at[2, j]], rj.at[dst], sem))
    for c in copies:
        c.wait()

    lanes = lax.iota(jnp.int32, _L)

    def group_body(g, loss_acc):
        ridx = g * _L + lanes

        def col_body(k, acc):
            cidx = jnp.full((_L,), 0, jnp.int32) + k
            gu = plsc.load_gather(ru, [ridx, cidx])
            gi = plsc.load_gather(ri, [ridx, cidx])
            gj = plsc.load_gather(rj, [ridx, cidx])
            return acc + gu * (gi - gj)

        z = lax.fori_loop(0, N_FACTORS, col_body, jnp.zeros((_L,), jnp.float32))
        e = jnp.exp(-jnp.abs(z))
        w = e / (2.0 + e)
        w2 = w * w
        ln1p = 2.0 * w * (1.0 + w2 * (
            (1.0 / 3.0) + w2 * ((1.0 / 5.0) + w2 * ((1.0 / 7.0) + w2 * (1.0 / 9.0)))))
        return loss_acc + jnp.maximum(-z, 0.0) + ln1p

    loss_acc = lax.fori_loop(0, _N_GROUPS, group_body, jnp.zeros((_L,), jnp.float32))
    total = jnp.sum(loss_acc)
    out_v[...] = jnp.zeros((_L,), jnp.float32) + total
    pltpu.sync_copy(out_v, out.at[wid])


_bpr_sc = functools.partial(
    pl.kernel,
    mesh=plsc.VectorSubcoreMesh(core_axis_name="c", subcore_axis_name="s"),
    compiler_params=pltpu.CompilerParams(
        needs_layout_passes=False, use_tc_tiling_on_sc=False),
    out_type=jax.ShapeDtypeStruct((_NW, _L), jnp.float32),
    scratch_types=[
        pltpu.VMEM((3, _N_CHUNKS, IDX_CHUNK), jnp.int32),
        pltpu.VMEM((_B_PER_W, N_FACTORS), jnp.float32),
        pltpu.VMEM((_B_PER_W, N_FACTORS), jnp.float32),
        pltpu.VMEM((_B_PER_W, N_FACTORS), jnp.float32),
        pltpu.VMEM((_L,), jnp.float32),
        pltpu.SemaphoreType.DMA,
    ],
)(_bpr_body)


def kernel(x, user_embeddings, item_embeddings):
    xr = x.astype(jnp.int32).reshape(3, BATCH // IDX_CHUNK, IDX_CHUNK)
    partials = _bpr_sc(xr, user_embeddings, item_embeddings)
    return jnp.sum(partials[:, 0])
